# baseline (device time: 109632 ns/iter reference)
import jax
import jax.numpy as jnp
from jax import lax
from jax.experimental import pallas as pl
from jax.experimental.pallas import tpu as pltpu

N_DEV = 8
BLK = 64


def kernel(x, Wq, K_ext, V_ext, Wo):
    B, Sq, Dm = x.shape
    _, Skv_loc, Hq, Dh = K_ext.shape
    nqb = Sq // BLK
    G = B * nqb * Hq
    bf16 = jnp.bfloat16

    Qf = jnp.dot(x.reshape(B * Sq, Dm).astype(bf16), Wq.astype(bf16),
                 preferred_element_type=bf16)
    q4 = (Qf.reshape(B, nqb, BLK, Hq, Dh)
          .transpose(0, 1, 3, 2, 4).reshape(G, BLK, Dh))
    Kb = (K_ext.astype(bf16).reshape(B, nqb, BLK, Hq, Dh)
          .transpose(0, 1, 3, 2, 4).reshape(G, BLK, Dh))
    Vb = (V_ext.astype(bf16).reshape(B, nqb, BLK, Hq, Dh)
          .transpose(0, 1, 3, 2, 4).reshape(G, BLK, Dh))
    kv = jnp.stack([Kb, Vb])

    def body(q_ref, kv_ref, out_ref, gather_ref, send_sems, recv_sems):
        me = lax.axis_index("i")
        left = lax.rem(me + N_DEV - 1, N_DEV)
        right = lax.rem(me + 1, N_DEV)

        barrier = pltpu.get_barrier_semaphore()
        for nbr in (left, right):
            pl.semaphore_signal(barrier, inc=1, device_id=(nbr,),
                                device_id_type=pl.DeviceIdType.MESH)
        pl.semaphore_wait(barrier, 2)

        gather_ref[me] = kv_ref[...]

        for h in range(N_DEV - 1):
            slot = lax.rem(me + N_DEV - h, N_DEV)
            rdma = pltpu.make_async_remote_copy(
                src_ref=gather_ref.at[slot],
                dst_ref=gather_ref.at[slot],
                send_sem=send_sems.at[h],
                recv_sem=recv_sems.at[h],
                device_id=(right,),
                device_id_type=pl.DeviceIdType.MESH,
            )
            rdma.start()
            rdma.wait()

        q = q_ref[...]
        scores = []
        for c in range(N_DEV):
            s = lax.dot_general(q, gather_ref[c, 0],
                                (((2,), (2,)), ((0,), (0,))),
                                preferred_element_type=jnp.float32)
            scores.append(s)
        S = jnp.concatenate(scores, axis=-1) * 0.125
        m = jnp.max(S, axis=-1, keepdims=True)
        w = jnp.exp(S - m)
        P = (w / jnp.sum(w, axis=-1, keepdims=True)).astype(bf16)
        acc = jnp.zeros((G, BLK, Dh), jnp.float32)
        for c in range(N_DEV):
            acc = acc + lax.dot_general(
                P[:, :, c * BLK:(c + 1) * BLK], gather_ref[c, 1],
                (((2,), (1,)), ((0,), (0,))),
                preferred_element_type=jnp.float32)
        out_ref[...] = acc

    ctx = pl.pallas_call(
        body,
        out_shape=jax.ShapeDtypeStruct((G, BLK, Dh), jnp.float32),
        in_specs=[pl.BlockSpec(memory_space=pltpu.VMEM),
                  pl.BlockSpec(memory_space=pltpu.VMEM)],
        out_specs=pl.BlockSpec(memory_space=pltpu.VMEM),
        scratch_shapes=[
            pltpu.VMEM((N_DEV, 2, G, BLK, Dh), bf16),
            pltpu.SemaphoreType.DMA((N_DEV - 1,)),
            pltpu.SemaphoreType.DMA((N_DEV - 1,)),
        ],
        compiler_params=pltpu.CompilerParams(collective_id=0),
    )(q4, kv)

    ctxf = (ctx.reshape(B, nqb, Hq, BLK, Dh)
            .transpose(0, 1, 3, 2, 4).reshape(B, Sq, Hq * Dh))
    return jnp.dot(ctxf.astype(bf16), Wo.astype(bf16),
                   preferred_element_type=jnp.float32)


# device time: 85663 ns/iter; 1.2798x vs baseline; 1.2798x over previous
import jax
import jax.numpy as jnp
from jax import lax
from jax.experimental import pallas as pl
from jax.experimental.pallas import tpu as pltpu

N_DEV = 8
BLK = 64


def kernel(x, Wq, K_ext, V_ext, Wo):
    B, Sq, Dm = x.shape
    _, Skv_loc, Hq, Dh = K_ext.shape
    nqb = Sq // BLK
    G = B * nqb * Hq
    bf16 = jnp.bfloat16

    Qf = jnp.dot(x.reshape(B * Sq, Dm).astype(bf16), Wq.astype(bf16),
                 preferred_element_type=bf16)
    q4 = (Qf.reshape(B, nqb, BLK, Hq, Dh)
          .transpose(0, 1, 3, 2, 4).reshape(G, BLK, Dh))
    Kb = (K_ext.astype(bf16).reshape(B, nqb, BLK, Hq, Dh)
          .transpose(0, 1, 3, 2, 4).reshape(G, BLK, Dh))
    Vb = (V_ext.astype(bf16).reshape(B, nqb, BLK, Hq, Dh)
          .transpose(0, 1, 3, 2, 4).reshape(G, BLK, Dh))
    kv = jnp.stack([Kb, Vb])

    def body(q_ref, kv_ref, out_ref, gather_ref, send_sems, recv_sems):
        me = lax.axis_index("i")

        gather_ref[me] = kv_ref[...]

        barrier = pltpu.get_barrier_semaphore()
        for j in range(1, N_DEV):
            peer = lax.rem(me + j, N_DEV)
            pl.semaphore_signal(barrier, inc=1, device_id=(peer,),
                                device_id_type=pl.DeviceIdType.MESH)
        pl.semaphore_wait(barrier, N_DEV - 1)

        sends = []
        for j in range(1, N_DEV):
            peer = lax.rem(me + j, N_DEV)
            rdma = pltpu.make_async_remote_copy(
                src_ref=gather_ref.at[me],
                dst_ref=gather_ref.at[me],
                send_sem=send_sems.at[j - 1],
                recv_sem=recv_sems.at[me],
                device_id=(peer,),
                device_id_type=pl.DeviceIdType.MESH,
            )
            rdma.start()
            sends.append(rdma)

        for j in range(1, N_DEV):
            origin = lax.rem(me + j, N_DEV)
            recv = pltpu.make_async_remote_copy(
                src_ref=gather_ref.at[origin],
                dst_ref=gather_ref.at[origin],
                send_sem=send_sems.at[j - 1],
                recv_sem=recv_sems.at[origin],
                device_id=(me,),
                device_id_type=pl.DeviceIdType.MESH,
            )
            recv.wait_recv()
        for rdma in sends:
            rdma.wait_send()

        q = q_ref[...]
        scores = []
        for c in range(N_DEV):
            s = lax.dot_general(q, gather_ref[c, 0],
                                (((2,), (2,)), ((0,), (0,))),
                                preferred_element_type=jnp.float32)
            scores.append(s)
        S = jnp.concatenate(scores, axis=-1) * 0.125
        m = jnp.max(S, axis=-1, keepdims=True)
        w = jnp.exp(S - m)
        P = (w / jnp.sum(w, axis=-1, keepdims=True)).astype(bf16)
        acc = jnp.zeros((G, BLK, Dh), jnp.float32)
        for c in range(N_DEV):
            acc = acc + lax.dot_general(
                P[:, :, c * BLK:(c + 1) * BLK], gather_ref[c, 1],
                (((2,), (1,)), ((0,), (0,))),
                preferred_element_type=jnp.float32)
        out_ref[...] = acc

    ctx = pl.pallas_call(
        body,
        out_shape=jax.ShapeDtypeStruct((G, BLK, Dh), jnp.float32),
        in_specs=[pl.BlockSpec(memory_space=pltpu.VMEM),
                  pl.BlockSpec(memory_space=pltpu.VMEM)],
        out_specs=pl.BlockSpec(memory_space=pltpu.VMEM),
        scratch_shapes=[
            pltpu.VMEM((N_DEV, 2, G, BLK, Dh), bf16),
            pltpu.SemaphoreType.DMA((N_DEV - 1,)),
            pltpu.SemaphoreType.DMA((N_DEV,)),
        ],
        compiler_params=pltpu.CompilerParams(collective_id=0),
    )(q4, kv)

    ctxf = (ctx.reshape(B, nqb, Hq, BLK, Dh)
            .transpose(0, 1, 3, 2, 4).reshape(B, Sq, Hq * Dh))
    return jnp.dot(ctxf.astype(bf16), Wo.astype(bf16),
                   preferred_element_type=jnp.float32)


# device time: 27474 ns/iter; 3.9904x vs baseline; 3.1180x over previous
import jax
import jax.numpy as jnp
from jax import lax
from jax.experimental import pallas as pl
from jax.experimental.pallas import tpu as pltpu

N_DEV = 8
BLK = 64
GPD = 4


def kernel(x, Wq, K_ext, V_ext, Wo):
    B, Sq, Dm = x.shape
    _, Skv_loc, Hq, Dh = K_ext.shape
    nqb = Sq // BLK
    G = B * nqb * Hq
    bf16 = jnp.bfloat16

    Qf = jnp.dot(x.reshape(B * Sq, Dm).astype(bf16), Wq.astype(bf16),
                 preferred_element_type=bf16)
    q4 = (Qf.reshape(B, nqb, BLK, Hq, Dh)
          .transpose(0, 1, 3, 2, 4).reshape(G, BLK, Dh))
    Kb = (K_ext.astype(bf16).reshape(B, nqb, BLK, Hq, Dh)
          .transpose(0, 1, 3, 2, 4).reshape(G, BLK, Dh))
    Vb = (V_ext.astype(bf16).reshape(B, nqb, BLK, Hq, Dh)
          .transpose(0, 1, 3, 2, 4).reshape(G, BLK, Dh))
    kv = jnp.stack([Kb, Vb], axis=1)

    def body(q_ref, kv_ref, out_ref, kvin_ref, ctx_ref,
             send1, recv1, send2, recv2):
        me = lax.axis_index("i")

        kvin_ref[me] = kv_ref[pl.ds(me * GPD, GPD)]

        barrier = pltpu.get_barrier_semaphore()
        for j in range(1, N_DEV):
            peer = lax.rem(me + j, N_DEV)
            pl.semaphore_signal(barrier, inc=1, device_id=(peer,),
                                device_id_type=pl.DeviceIdType.MESH)
        pl.semaphore_wait(barrier, N_DEV - 1)

        sends1 = []
        for j in range(1, N_DEV):
            peer = lax.rem(me + j, N_DEV)
            rdma = pltpu.make_async_remote_copy(
                src_ref=kv_ref.at[pl.ds(peer * GPD, GPD)],
                dst_ref=kvin_ref.at[me],
                send_sem=send1.at[j - 1],
                recv_sem=recv1.at[me],
                device_id=(peer,),
                device_id_type=pl.DeviceIdType.MESH,
            )
            rdma.start()
            sends1.append(rdma)
        for j in range(1, N_DEV):
            origin = lax.rem(me + j, N_DEV)
            recv = pltpu.make_async_remote_copy(
                src_ref=kvin_ref.at[origin], dst_ref=kvin_ref.at[origin],
                send_sem=send1.at[j - 1], recv_sem=recv1.at[origin],
                device_id=(me,), device_id_type=pl.DeviceIdType.MESH,
            )
            recv.wait_recv()

        qm = q_ref[pl.ds(me * GPD, GPD)]
        scores = []
        for c in range(N_DEV):
            s = lax.dot_general(qm, kvin_ref[c, :, 0],
                                (((2,), (2,)), ((0,), (0,))),
                                preferred_element_type=jnp.float32)
            scores.append(s)
        S = jnp.concatenate(scores, axis=-1) * 0.125
        m = jnp.max(S, axis=-1, keepdims=True)
        w = jnp.exp(S - m)
        P = (w / jnp.sum(w, axis=-1, keepdims=True)).astype(bf16)
        acc = jnp.zeros((GPD, BLK, Dh), jnp.float32)
        for c in range(N_DEV):
            acc = acc + lax.dot_general(
                P[:, :, c * BLK:(c + 1) * BLK], kvin_ref[c, :, 1],
                (((2,), (1,)), ((0,), (0,))),
                preferred_element_type=jnp.float32)
        ctx_ref[me] = acc.astype(bf16)

        sends2 = []
        for j in range(1, N_DEV):
            peer = lax.rem(me + j, N_DEV)
            rdma = pltpu.make_async_remote_copy(
                src_ref=ctx_ref.at[me], dst_ref=ctx_ref.at[me],
                send_sem=send2.at[j - 1], recv_sem=recv2.at[me],
                device_id=(peer,), device_id_type=pl.DeviceIdType.MESH,
            )
            rdma.start()
            sends2.append(rdma)
        for j in range(1, N_DEV):
            origin = lax.rem(me + j, N_DEV)
            recv = pltpu.make_async_remote_copy(
                src_ref=ctx_ref.at[origin], dst_ref=ctx_ref.at[origin],
                send_sem=send2.at[j - 1], recv_sem=recv2.at[origin],
                device_id=(me,), device_id_type=pl.DeviceIdType.MESH,
            )
            recv.wait_recv()
        for rdma in sends1 + sends2:
            rdma.wait_send()

        out_ref[...] = ctx_ref[...].reshape(G, BLK, Dh)

    ctx = pl.pallas_call(
        body,
        out_shape=jax.ShapeDtypeStruct((G, BLK, Dh), bf16),
        in_specs=[pl.BlockSpec(memory_space=pltpu.VMEM),
                  pl.BlockSpec(memory_space=pltpu.VMEM)],
        out_specs=pl.BlockSpec(memory_space=pltpu.VMEM),
        scratch_shapes=[
            pltpu.VMEM((N_DEV, GPD, 2, BLK, Dh), bf16),
            pltpu.VMEM((N_DEV, GPD, BLK, Dh), bf16),
            pltpu.SemaphoreType.DMA((N_DEV - 1,)),
            pltpu.SemaphoreType.DMA((N_DEV,)),
            pltpu.SemaphoreType.DMA((N_DEV - 1,)),
            pltpu.SemaphoreType.DMA((N_DEV,)),
        ],
        compiler_params=pltpu.CompilerParams(collective_id=0),
    )(q4, kv)

    ctxf = (ctx.reshape(B, nqb, Hq, BLK, Dh)
            .transpose(0, 1, 3, 2, 4).reshape(B, Sq, Hq * Dh))
    return jnp.dot(ctxf, Wo.astype(bf16), preferred_element_type=jnp.float32)


# device time: 21952 ns/iter; 4.9942x vs baseline; 1.2515x over previous
import jax
import jax.numpy as jnp
from jax import lax
from jax.experimental import pallas as pl
from jax.experimental.pallas import tpu as pltpu

N_DEV = 8
BLK = 64


def kernel(x, Wq, K_ext, V_ext, Wo):
    B, Sq, Dm = x.shape
    _, Skv_loc, Hq, Dh = K_ext.shape
    HD = Hq * Dh
    R = B * Sq
    bf16 = jnp.bfloat16

    x2 = x.reshape(R, Dm).astype(bf16)
    K2 = K_ext.reshape(R, HD).astype(bf16)
    V2 = V_ext.reshape(R, HD).astype(bf16)
    Wq2 = Wq.astype(bf16)
    Wo2 = Wo.astype(bf16)

    def body(x_ref, wq_ref, k_ref, v_ref, wo_ref, out_ref,
             kvin_ref, ctx_ref, send1, recv1, send2, recv2):
        me = lax.axis_index("i")
        row0 = me * BLK

        kvin_ref[me, 0] = k_ref[pl.ds(row0, BLK), :]
        kvin_ref[me, 1] = v_ref[pl.ds(row0, BLK), :]

        barrier = pltpu.get_barrier_semaphore()
        for j in range(1, N_DEV):
            peer = lax.rem(me + j, N_DEV)
            pl.semaphore_signal(barrier, inc=1, device_id=(peer,),
                                device_id_type=pl.DeviceIdType.MESH)
        pl.semaphore_wait(barrier, N_DEV - 1)

        sends1 = []
        for j in range(1, N_DEV):
            peer = lax.rem(me + j, N_DEV)
            for t, ref in ((0, k_ref), (1, v_ref)):
                rdma = pltpu.make_async_remote_copy(
                    src_ref=ref.at[pl.ds(peer * BLK, BLK), :],
                    dst_ref=kvin_ref.at[me, t],
                    send_sem=send1.at[2 * (j - 1) + t],
                    recv_sem=recv1.at[me, t],
                    device_id=(peer,),
                    device_id_type=pl.DeviceIdType.MESH,
                )
                rdma.start()
                sends1.append(rdma)

        qrow = lax.dot_general(x_ref[pl.ds(row0, BLK), :], wq_ref[...],
                               (((1,), (0,)), ((), ())),
                               preferred_element_type=jnp.float32
                               ).astype(jnp.bfloat16)

        for j in range(1, N_DEV):
            origin = lax.rem(me + j, N_DEV)
            for t in (0, 1):
                recv = pltpu.make_async_remote_copy(
                    src_ref=kvin_ref.at[origin, t],
                    dst_ref=kvin_ref.at[origin, t],
                    send_sem=send1.at[2 * (j - 1) + t],
                    recv_sem=recv1.at[origin, t],
                    device_id=(me,), device_id_type=pl.DeviceIdType.MESH,
                )
                recv.wait_recv()

        for h in range(Hq):
            cs = slice(h * Dh, (h + 1) * Dh)
            qh = qrow[:, cs]
            scores = []
            for c in range(N_DEV):
                s = lax.dot_general(qh, kvin_ref[c, 0, :, cs],
                                    (((1,), (1,)), ((), ())),
                                    preferred_element_type=jnp.float32)
                scores.append(s)
            S = jnp.concatenate(scores, axis=-1) * 0.125
            m = jnp.max(S, axis=-1, keepdims=True)
            w = jnp.exp(S - m)
            P = (w / jnp.sum(w, axis=-1, keepdims=True)).astype(jnp.bfloat16)
            acc = jnp.zeros((BLK, Dh), jnp.float32)
            for c in range(N_DEV):
                acc = acc + lax.dot_general(
                    P[:, c * BLK:(c + 1) * BLK], kvin_ref[c, 1, :, cs],
                    (((1,), (0,)), ((), ())),
                    preferred_element_type=jnp.float32)
            ctx_ref[pl.ds(row0, BLK), cs] = acc.astype(jnp.bfloat16)

        sends2 = []
        for j in range(1, N_DEV):
            peer = lax.rem(me + j, N_DEV)
            rdma = pltpu.make_async_remote_copy(
                src_ref=ctx_ref.at[pl.ds(row0, BLK), :],
                dst_ref=ctx_ref.at[pl.ds(row0, BLK), :],
                send_sem=send2.at[j - 1], recv_sem=recv2.at[me],
                device_id=(peer,), device_id_type=pl.DeviceIdType.MESH,
            )
            rdma.start()
            sends2.append(rdma)
        for j in range(1, N_DEV):
            origin = lax.rem(me + j, N_DEV)
            recv = pltpu.make_async_remote_copy(
                src_ref=ctx_ref.at[pl.ds(origin * BLK, BLK), :],
                dst_ref=ctx_ref.at[pl.ds(origin * BLK, BLK), :],
                send_sem=send2.at[j - 1], recv_sem=recv2.at[origin],
                device_id=(me,), device_id_type=pl.DeviceIdType.MESH,
            )
            recv.wait_recv()
        for rdma in sends1 + sends2:
            rdma.wait_send()

        out = lax.dot_general(ctx_ref[...], wo_ref[...],
                              (((1,), (0,)), ((), ())),
                              preferred_element_type=jnp.float32)
        out_ref[...] = out.reshape(B, Sq, Dm)

    return pl.pallas_call(
        body,
        out_shape=jax.ShapeDtypeStruct((B, Sq, Dm), jnp.float32),
        in_specs=[pl.BlockSpec(memory_space=pltpu.VMEM)] * 5,
        out_specs=pl.BlockSpec(memory_space=pltpu.VMEM),
        scratch_shapes=[
            pltpu.VMEM((N_DEV, 2, BLK, HD), bf16),
            pltpu.VMEM((R, HD), bf16),
            pltpu.SemaphoreType.DMA((2 * (N_DEV - 1),)),
            pltpu.SemaphoreType.DMA((N_DEV, 2)),
            pltpu.SemaphoreType.DMA((N_DEV - 1,)),
            pltpu.SemaphoreType.DMA((N_DEV,)),
        ],
        compiler_params=pltpu.CompilerParams(collective_id=0),
    )(x2, Wq2, K2, V2, Wo2)


# device time: 20685 ns/iter; 5.3001x vs baseline; 1.0613x over previous
import jax
import jax.numpy as jnp
from jax import lax
from jax.experimental import pallas as pl
from jax.experimental.pallas import tpu as pltpu

N_DEV = 8
BLK = 64


def kernel(x, Wq, K_ext, V_ext, Wo):
    B, Sq, Dm = x.shape
    _, Skv_loc, Hq, Dh = K_ext.shape
    HD = Hq * Dh
    R = B * Sq
    bf16 = jnp.bfloat16

    x2 = x.reshape(R, Dm)
    K3 = K_ext.reshape(R, Hq, Dh)
    V3 = V_ext.reshape(R, Hq, Dh)

    def body(x_ref, wq_ref, k_ref, v_ref, wo_ref, out_ref,
             kbf_ref, vbf_ref, kvin_ref, ctx_ref,
             send1, recv1, send2, recv2):
        me = lax.axis_index("i")
        row0 = me * BLK

        barrier = pltpu.get_barrier_semaphore()
        for j in range(1, N_DEV):
            peer = lax.rem(me + j, N_DEV)
            pl.semaphore_signal(barrier, inc=1, device_id=(peer,),
                                device_id_type=pl.DeviceIdType.MESH)

        def relayout_block(r):
            rows = pl.ds(r * BLK, BLK)
            for t, src, dst in ((0, k_ref, kbf_ref), (1, v_ref, vbf_ref)):
                for h in range(Hq):
                    dst[rows, h * Dh:(h + 1) * Dh] = (
                        src[rows, h, :].astype(bf16))

        relayout_block(me)
        kvin_ref[me, 0] = kbf_ref[pl.ds(row0, BLK), :]
        kvin_ref[me, 1] = vbf_ref[pl.ds(row0, BLK), :]

        pl.semaphore_wait(barrier, N_DEV - 1)

        sends1 = []
        for j in range(1, N_DEV):
            peer = lax.rem(me + j, N_DEV)
            relayout_block(peer)
            for t, src in ((0, kbf_ref), (1, vbf_ref)):
                rdma = pltpu.make_async_remote_copy(
                    src_ref=src.at[pl.ds(peer * BLK, BLK), :],
                    dst_ref=kvin_ref.at[me, t],
                    send_sem=send1.at[2 * (j - 1) + t],
                    recv_sem=recv1.at[me, t],
                    device_id=(peer,),
                    device_id_type=pl.DeviceIdType.MESH,
                )
                rdma.start()
                sends1.append(rdma)

        qrow = lax.dot_general(x_ref[pl.ds(row0, BLK), :].astype(bf16),
                               wq_ref[...].astype(bf16),
                               (((1,), (0,)), ((), ())),
                               preferred_element_type=jnp.float32
                               ).astype(bf16)

        for j in range(1, N_DEV):
            origin = lax.rem(me + j, N_DEV)
            for t in (0, 1):
                recv = pltpu.make_async_remote_copy(
                    src_ref=kvin_ref.at[origin, t],
                    dst_ref=kvin_ref.at[origin, t],
                    send_sem=send1.at[2 * (j - 1) + t],
                    recv_sem=recv1.at[origin, t],
                    device_id=(me,), device_id_type=pl.DeviceIdType.MESH,
                )
                recv.wait_recv()

        for h in range(Hq):
            cs = slice(h * Dh, (h + 1) * Dh)
            qh = qrow[:, cs]
            scores = []
            for c in range(N_DEV):
                s = lax.dot_general(qh, kvin_ref[c, 0, :, cs],
                                    (((1,), (1,)), ((), ())),
                                    preferred_element_type=jnp.float32)
                scores.append(s)
            S = jnp.concatenate(scores, axis=-1) * 0.125
            m = jnp.max(S, axis=-1, keepdims=True)
            w = jnp.exp(S - m)
            P = (w / jnp.sum(w, axis=-1, keepdims=True)).astype(bf16)
            acc = jnp.zeros((BLK, Dh), jnp.float32)
            for c in range(N_DEV):
                acc = acc + lax.dot_general(
                    P[:, c * BLK:(c + 1) * BLK], kvin_ref[c, 1, :, cs],
                    (((1,), (0,)), ((), ())),
                    preferred_element_type=jnp.float32)
            ctx_ref[pl.ds(row0, BLK), cs] = acc.astype(bf16)

        sends2 = []
        for j in range(1, N_DEV):
            peer = lax.rem(me + j, N_DEV)
            rdma = pltpu.make_async_remote_copy(
                src_ref=ctx_ref.at[pl.ds(row0, BLK), :],
                dst_ref=ctx_ref.at[pl.ds(row0, BLK), :],
                send_sem=send2.at[j - 1], recv_sem=recv2.at[me],
                device_id=(peer,), device_id_type=pl.DeviceIdType.MESH,
            )
            rdma.start()
            sends2.append(rdma)
        for j in range(1, N_DEV):
            origin = lax.rem(me + j, N_DEV)
            recv = pltpu.make_async_remote_copy(
                src_ref=ctx_ref.at[pl.ds(origin * BLK, BLK), :],
                dst_ref=ctx_ref.at[pl.ds(origin * BLK, BLK), :],
                send_sem=send2.at[j - 1], recv_sem=recv2.at[origin],
                device_id=(me,), device_id_type=pl.DeviceIdType.MESH,
            )
            recv.wait_recv()
        for rdma in sends1 + sends2:
            rdma.wait_send()

        out = lax.dot_general(ctx_ref[...], wo_ref[...].astype(bf16),
                              (((1,), (0,)), ((), ())),
                              preferred_element_type=jnp.float32)
        out_ref[...] = out.astype(bf16).reshape(B, Sq, Dm)

    return pl.pallas_call(
        body,
        out_shape=jax.ShapeDtypeStruct((B, Sq, Dm), bf16),
        in_specs=[pl.BlockSpec(memory_space=pltpu.VMEM)] * 5,
        out_specs=pl.BlockSpec(memory_space=pltpu.VMEM),
        scratch_shapes=[
            pltpu.VMEM((R, HD), bf16),
            pltpu.VMEM((R, HD), bf16),
            pltpu.VMEM((N_DEV, 2, BLK, HD), bf16),
            pltpu.VMEM((R, HD), bf16),
            pltpu.SemaphoreType.DMA((2 * (N_DEV - 1),)),
            pltpu.SemaphoreType.DMA((N_DEV, 2)),
            pltpu.SemaphoreType.DMA((N_DEV - 1,)),
            pltpu.SemaphoreType.DMA((N_DEV,)),
        ],
        compiler_params=pltpu.CompilerParams(collective_id=0),
    )(x2, Wq, K3, V3, Wo)
